# Initial kernel scaffold; baseline (speedup 1.0000x reference)
import jax
import jax.numpy as jnp
from jax.experimental import pallas as pl


def kernel(x, edge_index, W1, b1, W2, b2, Wt, bt, We, be):
    raise NotImplementedError("write your pallas kernel here")


# speedup vs baseline: 31.8423x; 31.8423x over previous
import functools

import jax
import jax.numpy as jnp
from jax import lax
from jax.experimental import pallas as pl
from jax.experimental.pallas import tpu as pltpu
from jax.experimental.pallas import tpu_sc as plsc

N = 10000
NPAD = 10240
D = 128
E = 320000
NC = 2
NS = 16
CHUNK = 128
NCHUNK = 79
EPAD = NC * NS * NCHUNK * CHUNK
ROWS_PER_TILE = NPAD // NS

_ZR = 16


@functools.cache
def _sc_kernels():
  mesh = plsc.VectorSubcoreMesh(
      core_axis_name="c", subcore_axis_name="s",
      num_cores=NC, num_subcores=NS)

  @functools.partial(
      pl.kernel,
      out_type=jax.ShapeDtypeStruct((NC, NPAD), jnp.float32),
      mesh=mesh,
      scratch_types=[
          pltpu.VMEM((NCHUNK, CHUNK), jnp.int32),
          pltpu.VMEM((CHUNK,), jnp.float32),
          pltpu.VMEM((ROWS_PER_TILE,), jnp.float32),
          pltpu.VMEM_SHARED((NPAD,), jnp.float32),
      ],
  )
  def _sc_deg(dst_hbm, deg_out, dstv, onesv, zv, sdeg):
    cid = lax.axis_index("c")
    sid = lax.axis_index("s")
    wid = sid * NC + cid

    def fill(i, _):
      onesv[pl.ds(i * 16, 16)] = jnp.ones((16,), jnp.float32)
      return 0
    lax.fori_loop(0, CHUNK // 16, fill, 0)

    def zfill(i, _):
      zv[pl.ds(i * 16, 16)] = jnp.zeros((16,), jnp.float32)
      return 0
    lax.fori_loop(0, ROWS_PER_TILE // 16, zfill, 0)

    pltpu.sync_copy(zv, sdeg.at[pl.ds(sid * ROWS_PER_TILE, ROWS_PER_TILE)])
    pltpu.sync_copy(dst_hbm.at[wid], dstv)
    plsc.subcore_barrier()

    def body(j, _):
      pltpu.sync_copy(onesv, sdeg.at[dstv.at[j]], add=True)
      return 0
    lax.fori_loop(0, NCHUNK, body, 0)

    plsc.subcore_barrier()
    pltpu.sync_copy(sdeg.at[pl.ds(sid * ROWS_PER_TILE, ROWS_PER_TILE)],
                    deg_out.at[cid, pl.ds(sid * ROWS_PER_TILE, ROWS_PER_TILE)])

  @functools.partial(
      pl.kernel,
      out_type=jax.ShapeDtypeStruct((NC, NPAD, D), jnp.float32),
      mesh=mesh,
      scratch_types=[
          pltpu.VMEM((NCHUNK, CHUNK), jnp.int32),
          pltpu.VMEM((2, CHUNK), jnp.int32),
          pltpu.VMEM((2, CHUNK, D), jnp.float32),
          pltpu.VMEM((_ZR, D), jnp.float32),
          pltpu.VMEM_SHARED((NPAD, D), jnp.float32),
          pltpu.SemaphoreType.DMA,
          pltpu.SemaphoreType.DMA,
      ],
  )
  def _sc_agg(u_hbm, src_hbm, dst_hbm, agg_out, dstv, srcb, rows, zbuf, accum,
              gsem, ssem):
    cid = lax.axis_index("c")
    sid = lax.axis_index("s")
    wid = sid * NC + cid

    def zfill(i, _):
      zbuf[i // 8, pl.ds((i % 8) * 16, 16)] = jnp.zeros((16,), jnp.float32)
      return 0
    lax.fori_loop(0, _ZR * D // 16, zfill, 0)

    n_zcopy = ROWS_PER_TILE // _ZR
    def zcopy(k, _):
      pltpu.sync_copy(zbuf,
                      accum.at[pl.ds(sid * ROWS_PER_TILE + k * _ZR, _ZR)])
      return 0
    lax.fori_loop(0, n_zcopy, zcopy, 0)

    pltpu.sync_copy(dst_hbm.at[wid], dstv)
    plsc.subcore_barrier()

    pltpu.sync_copy(src_hbm.at[wid, 0], srcb.at[0])
    pltpu.async_copy(u_hbm.at[srcb.at[0]], rows.at[0], gsem)

    def body(j, _):
      b = j % 2
      nb = (j + 1) % 2
      pltpu.sync_copy(src_hbm.at[wid, j + 1], srcb.at[nb])

      @pl.when(j >= 1)
      def _():
        pltpu.make_async_copy(u_hbm.at[pl.ds(0, CHUNK)], rows.at[nb],
                              ssem).wait()
      pltpu.async_copy(u_hbm.at[srcb.at[nb]], rows.at[nb], gsem)
      pltpu.make_async_copy(u_hbm.at[srcb.at[b]], rows.at[b], gsem).wait()
      pltpu.async_copy(rows.at[b], accum.at[dstv.at[j]], ssem, add=True)
      return 0
    lax.fori_loop(0, NCHUNK - 1, body, 0)

    lastb = (NCHUNK - 1) % 2
    pltpu.make_async_copy(u_hbm.at[srcb.at[lastb]], rows.at[lastb],
                          gsem).wait()
    pltpu.make_async_copy(u_hbm.at[pl.ds(0, CHUNK)], rows.at[1 - lastb],
                          ssem).wait()
    pltpu.sync_copy(rows.at[lastb], accum.at[dstv.at[NCHUNK - 1]], add=True)

    plsc.subcore_barrier()
    pltpu.sync_copy(
        accum.at[pl.ds(sid * ROWS_PER_TILE, ROWS_PER_TILE)],
        agg_out.at[cid, pl.ds(sid * ROWS_PER_TILE, ROWS_PER_TILE)])

  return _sc_deg, _sc_agg


_RB = 1024
_NROWB = NPAD // _RB


def _tc1_body(deg_ref, x_ref, w_ref, u_ref):
  d = deg_ref[0] + deg_ref[1] + 1.0
  dinv = lax.rsqrt(d)
  u_ref[...] = dinv * jnp.dot(x_ref[...], w_ref[...],
                              preferred_element_type=jnp.float32)


def _tc_mid_body(deg_ref, p_ref, u_ref, b_ref, w_ref, o_ref):
  dinv = lax.rsqrt(deg_ref[0] + deg_ref[1] + 1.0)
  h = jnp.maximum(dinv * (p_ref[0] + p_ref[1] + u_ref[...]) + b_ref[...], 0.0)
  o_ref[...] = dinv * jnp.dot(h, w_ref[...],
                              preferred_element_type=jnp.float32)


def _tc_head_body(deg_ref, p_ref, u_ref, b_ref, w_ref, bh_ref, o_ref):
  dinv = lax.rsqrt(deg_ref[0] + deg_ref[1] + 1.0)
  h = jnp.maximum(dinv * (p_ref[0] + p_ref[1] + u_ref[...]) + b_ref[...], 0.0)
  o_ref[...] = jnp.dot(h, w_ref[...],
                       preferred_element_type=jnp.float32) + bh_ref[...]


_deg_spec = pl.BlockSpec((NC, _RB, 1), lambda i: (0, i, 0))
_row_spec = pl.BlockSpec((_RB, D), lambda i: (i, 0))
_p_spec = pl.BlockSpec((NC, _RB, D), lambda i: (0, i, 0))
_w_spec = pl.BlockSpec((D, D), lambda i: (0, 0))
_b_spec = pl.BlockSpec((1, D), lambda i: (0, 0))
_f32 = jnp.float32


def _tc1(degc, x_pad, W1):
  return pl.pallas_call(
      _tc1_body,
      grid=(_NROWB,),
      in_specs=[_deg_spec, _row_spec, _w_spec],
      out_specs=_row_spec,
      out_shape=jax.ShapeDtypeStruct((NPAD, D), _f32),
  )(degc, x_pad, W1)


def _tc_mid(degc, parts, u, b, W):
  return pl.pallas_call(
      _tc_mid_body,
      grid=(_NROWB,),
      in_specs=[_deg_spec, _p_spec, _row_spec, _b_spec, _w_spec],
      out_specs=_row_spec,
      out_shape=jax.ShapeDtypeStruct((NPAD, D), _f32),
  )(degc, parts, u, b, W)


def _tc_head(degc, parts, u, b, Wh, bh):
  return pl.pallas_call(
      _tc_head_body,
      grid=(_NROWB,),
      in_specs=[_deg_spec, _p_spec, _row_spec, _b_spec, _w_spec, _b_spec],
      out_specs=_row_spec,
      out_shape=jax.ShapeDtypeStruct((NPAD, D), _f32),
  )(degc, parts, u, b, Wh, bh)


@jax.jit
def kernel(x, edge_index, W1, b1, W2, b2, Wt, bt, We, be):
  src = edge_index[0].astype(jnp.int32)
  dst = edge_index[1].astype(jnp.int32)

  npad_e = EPAD - E
  pad_idx = N + (jnp.arange(npad_e, dtype=jnp.int32) % (NPAD - N))
  src_t = jnp.concatenate([src, pad_idx]).reshape(NC * NS, NCHUNK, CHUNK)
  dst_t = jnp.concatenate([dst, pad_idx]).reshape(NC * NS, NCHUNK, CHUNK)

  x_pad = jnp.zeros((NPAD, D), _f32).at[:N].set(x)

  sc_deg, sc_agg = _sc_kernels()
  deg_part = sc_deg(dst_t)
  degc = deg_part.reshape(NC, NPAD, 1)

  b1r = b1.reshape(1, D)
  b2r = b2.reshape(1, D)
  Wh = jnp.zeros((D, D), _f32).at[:, 0:1].set(Wt).at[:, 1:2].set(We)
  bh = jnp.zeros((1, D), _f32).at[0, 0].set(bt[0]).at[0, 1].set(be[0])

  u1 = _tc1(degc, x_pad, W1)
  p1 = sc_agg(u1, src_t, dst_t)
  u2 = _tc_mid(degc, p1, u1, b1r, W2)
  p2 = sc_agg(u2, src_t, dst_t)
  out = _tc_head(degc, p2, u2, b2r, Wh, bh)

  return (out[:N, 0:1], out[:N, 1:2])
